# Initial kernel scaffold; baseline (speedup 1.0000x reference)
import jax
import jax.numpy as jnp
from jax.experimental import pallas as pl


def kernel(tokens, embed_weight):
    raise NotImplementedError("write your pallas kernel here")


# speedup vs baseline: 1.2903x; 1.2903x over previous
import functools

import jax
import jax.numpy as jnp
from jax import lax
from jax.experimental import pallas as pl
from jax.experimental.pallas import tpu as pltpu
from jax.experimental.pallas import tpu_sc as plsc

BATCH = 64
SEQ = 2048
D = 512
N_TOK = BATCH * SEQ
NC = 2
NS = 16
NW = NC * NS
TOK_PER_W = N_TOK // NW
CHUNK = 128
N_CHUNKS = TOK_PER_W // CHUNK


@functools.partial(
    pl.kernel,
    mesh=plsc.VectorSubcoreMesh(core_axis_name="c", subcore_axis_name="s"),
    out_type=jax.ShapeDtypeStruct((N_TOK, D), jnp.float32),
    scratch_types=[
        pltpu.VMEM((TOK_PER_W,), jnp.int32),
        pltpu.VMEM((CHUNK, D), jnp.float32),
        pltpu.SemaphoreType.DMA,
    ],
)
def _embed_lookup(tokens_hbm, table_hbm, out_hbm, idx_v, rows_v, sem):
    wid = lax.axis_index("s") * NC + lax.axis_index("c")
    base = wid * TOK_PER_W
    pltpu.sync_copy(tokens_hbm.at[pl.ds(base, TOK_PER_W)], idx_v)

    def body(c, carry):
        off = c * CHUNK
        pltpu.async_copy(
            table_hbm.at[idx_v.at[pl.ds(off, CHUNK)]], rows_v, sem
        ).wait()
        pltpu.sync_copy(rows_v, out_hbm.at[pl.ds(base + off, CHUNK)])
        return carry

    lax.fori_loop(0, N_CHUNKS, body, 0)


def kernel(tokens, embed_weight):
    flat = tokens.reshape(-1).astype(jnp.int32)
    out = _embed_lookup(flat, embed_weight)
    return out.reshape(BATCH, SEQ, D)


# speedup vs baseline: 1.2941x; 1.0030x over previous
import functools

import jax
import jax.numpy as jnp
from jax import lax
from jax.experimental import pallas as pl
from jax.experimental.pallas import tpu as pltpu
from jax.experimental.pallas import tpu_sc as plsc

BATCH = 64
SEQ = 2048
D = 512
N_TOK = BATCH * SEQ
NC = 2
NS = 16
NW = NC * NS
TOK_PER_W = N_TOK // NW
CHUNK = 64
N_CHUNKS = TOK_PER_W // CHUNK


@functools.partial(
    pl.kernel,
    mesh=plsc.VectorSubcoreMesh(core_axis_name="c", subcore_axis_name="s"),
    out_type=jax.ShapeDtypeStruct((N_TOK, D), jnp.float32),
    scratch_types=[
        pltpu.VMEM((TOK_PER_W,), jnp.int32),
        pltpu.VMEM((CHUNK, D), jnp.float32),
        pltpu.VMEM((CHUNK, D), jnp.float32),
        pltpu.SemaphoreType.DMA,
        pltpu.SemaphoreType.DMA,
        pltpu.SemaphoreType.DMA,
        pltpu.SemaphoreType.DMA,
    ],
)
def _embed_lookup(tokens_hbm, table_hbm, out_hbm, idx_v, buf0, buf1,
                  sg0, sg1, ss0, ss1):
    wid = lax.axis_index("s") * NC + lax.axis_index("c")
    base = wid * TOK_PER_W
    pltpu.sync_copy(tokens_hbm.at[pl.ds(base, TOK_PER_W)], idx_v)

    def g_start(c, buf, sem):
        pltpu.async_copy(
            table_hbm.at[idx_v.at[pl.ds(c * CHUNK, CHUNK)]], buf, sem)

    def g_wait(buf, sem):
        pltpu.make_async_copy(
            table_hbm.at[idx_v.at[pl.ds(0, CHUNK)]], buf, sem).wait()

    def s_start(c, buf, sem):
        pltpu.async_copy(buf, out_hbm.at[pl.ds(base + c * CHUNK, CHUNK)], sem)

    def s_wait(buf, sem):
        pltpu.make_async_copy(buf, out_hbm.at[pl.ds(base, CHUNK)], sem).wait()

    g_start(0, buf0, sg0)
    g_start(1, buf1, sg1)

    def body(g, carry):
        c0 = 2 * g
        g_wait(buf0, sg0)
        s_start(c0, buf0, ss0)
        g_wait(buf1, sg1)
        s_start(c0 + 1, buf1, ss1)
        s_wait(buf0, ss0)
        g_start(c0 + 2, buf0, sg0)
        s_wait(buf1, ss1)
        g_start(c0 + 3, buf1, sg1)
        return carry

    lax.fori_loop(0, N_CHUNKS // 2 - 1, body, 0)

    c0 = N_CHUNKS - 2
    g_wait(buf0, sg0)
    s_start(c0, buf0, ss0)
    g_wait(buf1, sg1)
    s_start(c0 + 1, buf1, ss1)
    s_wait(buf0, ss0)
    s_wait(buf1, ss1)


def kernel(tokens, embed_weight):
    flat = tokens.reshape(-1).astype(jnp.int32)
    out = _embed_lookup(flat, embed_weight)
    return out.reshape(BATCH, SEQ, D)
